# baseline (speedup 1.0000x reference)
import functools

import jax
import jax.numpy as jnp
from jax import lax
from jax.experimental import pallas as pl
from jax.experimental.pallas import tpu as pltpu
from jax.experimental.pallas import tpu_sc as plsc

N = 4096
D = 1024
DFF = 2048
E = 8
TOPK = 2
TM = 256
TMF = 384
NB = 29

P = NB * TMF
NTB = N // TM
NW = 32
TPW = N // NW
CH = 32
NCH = TPW // CH
EPS = 1e-6


def _rms_rows(x, w):
    return x * lax.rsqrt(jnp.mean(x * x, axis=-1, keepdims=True) + EPS) * w


def _inproj_body(s_ref, win_ref, inw_ref, wr_ref, h_ref, log_ref):
    h = lax.dot_general(s_ref[...], win_ref[...], (((1,), (1,)), ((), ())),
                        preferred_element_type=jnp.float32)
    h = _rms_rows(h, inw_ref[...])
    h_ref[...] = h
    log_ref[...] = lax.dot_general(h, wr_ref[...], (((1,), (1,)), ((), ())),
                                   preferred_element_type=jnp.float32)


def _inproj(s, W_in, inw2, W_router):
    return pl.pallas_call(
        _inproj_body,
        grid=(NTB,),
        in_specs=[
            pl.BlockSpec((TM, D), lambda i: (i, 0)),
            pl.BlockSpec((D, D), lambda i: (0, 0)),
            pl.BlockSpec((1, D), lambda i: (0, 0)),
            pl.BlockSpec((E, D), lambda i: (0, 0)),
        ],
        out_specs=[
            pl.BlockSpec((TM, D), lambda i: (i, 0)),
            pl.BlockSpec((TM, E), lambda i: (i, 0)),
        ],
        out_shape=[
            jax.ShapeDtypeStruct((N, D), jnp.float32),
            jax.ShapeDtypeStruct((N, E), jnp.float32),
        ],
    )(s, W_in, inw2, W_router)


def _router_body(log_ref, bias_ref, gates_ref, dst0_ref, dst1_ref, be_ref,
                 nbu_ref):
    logits = log_ref[...]
    biased = logits + bias_ref[...]
    iota_e = lax.broadcasted_iota(jnp.int32, (N, E), 1).astype(jnp.float32)
    max0 = jnp.max(biased, axis=1, keepdims=True)
    idx0 = jnp.min(jnp.where(biased == max0, iota_e, jnp.float32(E)),
                   axis=1, keepdims=True)
    masked = jnp.where(iota_e == idx0, -jnp.inf, biased)
    max1 = jnp.max(masked, axis=1, keepdims=True)
    idx1 = jnp.min(jnp.where(masked == max1, iota_e, jnp.float32(E)),
                   axis=1, keepdims=True)
    sel0 = jnp.sum(jnp.where(iota_e == idx0, logits, 0.0), axis=1, keepdims=True)
    sel1 = jnp.sum(jnp.where(iota_e == idx1, logits, 0.0), axis=1, keepdims=True)
    m = jnp.maximum(sel0, sel1)
    e0 = jnp.exp(sel0 - m)
    e1 = jnp.exp(sel1 - m)
    tot = e0 + e1
    g0 = e0 / tot
    g1 = e1 / tot
    gates_ref[...] = jnp.concatenate(
        [g0, g1, jnp.zeros((N, E - TOPK), jnp.float32)], axis=1)

    oh0 = (iota_e == idx0).astype(jnp.float32)
    oh1 = (iota_e == idx1).astype(jnp.float32)
    ohp = jnp.concatenate([oh0, oh1], axis=0)
    x = ohp
    sh = 1
    while sh < 2 * N:
        x = x + jnp.concatenate(
            [jnp.zeros((sh, E), jnp.float32), x[:2 * N - sh]], axis=0)
        sh *= 2
    excl = x - ohp
    counts = x[2 * N - 1:2 * N, :]
    blocks_e = jnp.floor((counts + (TMF - 1)) / TMF)
    b = blocks_e
    for shw in (1, 2, 4):
        b = b + jnp.concatenate(
            [jnp.zeros((1, shw), jnp.float32), b[:, :E - shw]], axis=1)
    bstart = b - blocks_e
    dstcol = excl + bstart * TMF
    dst_p = jnp.sum(ohp * dstcol, axis=1, keepdims=True).astype(jnp.int32)
    dst0_ref[...] = dst_p[:N]
    dst1_ref[...] = dst_p[N:]
    biot = lax.broadcasted_iota(jnp.int32, (NB, E), 0).astype(jnp.float32)
    be = jnp.sum((biot >= bstart).astype(jnp.float32), axis=1, keepdims=True) - 1.0
    be_ref[...] = jnp.clip(be, 0.0, float(E - 1)).astype(jnp.int32)
    nbu_ref[...] = jnp.sum(blocks_e, axis=1, keepdims=True).astype(jnp.int32)


def _router(logits, bias2):
    return pl.pallas_call(
        _router_body,
        grid=(1,),
        in_specs=[
            pl.BlockSpec((N, E), lambda i: (0, 0)),
            pl.BlockSpec((1, E), lambda i: (0, 0)),
        ],
        out_specs=[
            pl.BlockSpec((N, E), lambda i: (0, 0)),
            pl.BlockSpec((N, 1), lambda i: (0, 0)),
            pl.BlockSpec((N, 1), lambda i: (0, 0)),
            pl.BlockSpec((NB, 1), lambda i: (0, 0)),
            pl.BlockSpec((1, 1), lambda i: (0, 0)),
        ],
        out_shape=[
            jax.ShapeDtypeStruct((N, E), jnp.float32),
            jax.ShapeDtypeStruct((N, 1), jnp.int32),
            jax.ShapeDtypeStruct((N, 1), jnp.int32),
            jax.ShapeDtypeStruct((NB, 1), jnp.int32),
            jax.ShapeDtypeStruct((1, 1), jnp.int32),
        ],
    )(logits, bias2)


def _sc_scatter_body(h_hbm, dst0_hbm, dst1_hbm, out_hbm, rows0_v, rows1_v,
                     idx0_v, idx1_v, rsem, wsem0, wsem1):
    wid = lax.axis_index("s") * 2 + lax.axis_index("c")
    pltpu.sync_copy(dst0_hbm.at[wid], idx0_v)
    pltpu.sync_copy(dst1_hbm.at[wid], idx1_v)
    bufs = (rows0_v, rows1_v)
    rd = [None] * NCH
    wr = [None] * NCH
    rd[0] = pltpu.async_copy(h_hbm.at[pl.ds(wid * TPW, CH)], bufs[0], rsem)
    for j in range(NCH):
        rd[j].wait()
        buf = bufs[j % 2]
        wsem = (wsem0, wsem1)[j % 2]
        wr[j] = (pltpu.async_copy(buf, out_hbm.at[idx0_v.at[j]], wsem),
                 pltpu.async_copy(buf, out_hbm.at[idx1_v.at[j]], wsem))
        if j + 1 < NCH:
            if j >= 1:
                wr[j - 1][0].wait()
                wr[j - 1][1].wait()
            t0 = wid * TPW + (j + 1) * CH
            rd[j + 1] = pltpu.async_copy(h_hbm.at[pl.ds(t0, CH)],
                                         bufs[(j + 1) % 2], rsem)
    wr[NCH - 2][0].wait()
    wr[NCH - 2][1].wait()
    wr[NCH - 1][0].wait()
    wr[NCH - 1][1].wait()


def _sc_scatter(h, dst0_3d, dst1_3d):
    mesh = plsc.VectorSubcoreMesh(core_axis_name="c", subcore_axis_name="s",
                                  num_cores=2, num_subcores=16)
    fn = pl.kernel(
        _sc_scatter_body,
        out_type=jax.ShapeDtypeStruct((P, D), jnp.float32),
        mesh=mesh,
        scratch_types=[
            pltpu.VMEM((CH, D), jnp.float32),
            pltpu.VMEM((CH, D), jnp.float32),
            pltpu.VMEM((NCH, CH), jnp.int32),
            pltpu.VMEM((NCH, CH), jnp.int32),
            pltpu.SemaphoreType.DMA,
            pltpu.SemaphoreType.DMA,
            pltpu.SemaphoreType.DMA,
        ],
    )
    return fn(h, dst0_3d, dst1_3d)


def _ffn_body(be_ref, nbu_ref, h_ref, wg_ref, wu_ref, wd_ref, enw_ref, y_ref):
    @pl.when(pl.program_id(0) < nbu_ref[0])
    def _():
        hb = h_ref[...]
        g = lax.dot_general(hb, wg_ref[0], (((1,), (1,)), ((), ())),
                            preferred_element_type=jnp.float32)
        u = lax.dot_general(hb, wu_ref[0], (((1,), (1,)), ((), ())),
                            preferred_element_type=jnp.float32)
        act = g * jax.nn.sigmoid(g) * u
        dd = lax.dot_general(act, wd_ref[0], (((1,), (1,)), ((), ())),
                             preferred_element_type=jnp.float32)
        y_ref[...] = _rms_rows(dd, enw_ref[0])


def _ffn(be, nbu, h_sorted, Wg, Wu, Wd, exp_norm_w):
    grid_spec = pltpu.PrefetchScalarGridSpec(
        num_scalar_prefetch=2,
        grid=(NB,),
        in_specs=[
            pl.BlockSpec((TMF, D), lambda b, eref, nref: (b, 0)),
            pl.BlockSpec((1, DFF, D), lambda b, eref, nref: (eref[b], 0, 0)),
            pl.BlockSpec((1, DFF, D), lambda b, eref, nref: (eref[b], 0, 0)),
            pl.BlockSpec((1, D, DFF), lambda b, eref, nref: (eref[b], 0, 0)),
            pl.BlockSpec((1, 1, D), lambda b, eref, nref: (eref[b], 0, 0)),
        ],
        out_specs=pl.BlockSpec((TMF, D), lambda b, eref, nref: (b, 0)),
    )
    return pl.pallas_call(
        _ffn_body,
        grid_spec=grid_spec,
        out_shape=jax.ShapeDtypeStruct((P, D), jnp.float32),
        compiler_params=pltpu.CompilerParams(
            dimension_semantics=("arbitrary",)),
    )(be, nbu, h_sorted, Wg, Wu, Wd, exp_norm_w.reshape(E, 1, D))


def _sc_gather_body(y_hbm, dstp_hbm, out_hbm, rows0_v, rows1_v, idx_v,
                    rsem, wsem0, wsem1):
    wid = lax.axis_index("s") * 2 + lax.axis_index("c")
    pltpu.sync_copy(dstp_hbm.at[wid], idx_v)
    nch = 2 * NCH
    bufs = (rows0_v, rows1_v)
    rd = [None] * nch
    wr = [None] * nch
    rd[0] = pltpu.async_copy(y_hbm.at[idx_v.at[0]], bufs[0], rsem)
    for j in range(nch):
        rd[j].wait()
        buf = bufs[j % 2]
        wr[j] = pltpu.async_copy(
            buf, out_hbm.at[pl.ds(wid * 2 * TPW + j * CH, CH)],
            (wsem0, wsem1)[j % 2])
        if j + 1 < nch:
            if j >= 1:
                wr[j - 1].wait()
            rd[j + 1] = pltpu.async_copy(y_hbm.at[idx_v.at[j + 1]],
                                         bufs[(j + 1) % 2], rsem)
    wr[nch - 2].wait()
    wr[nch - 1].wait()


def _sc_gather(y_sorted, dstp_3d):
    mesh = plsc.VectorSubcoreMesh(core_axis_name="c", subcore_axis_name="s",
                                  num_cores=2, num_subcores=16)
    fn = pl.kernel(
        _sc_gather_body,
        out_type=jax.ShapeDtypeStruct((TOPK * N, D), jnp.float32),
        mesh=mesh,
        scratch_types=[
            pltpu.VMEM((CH, D), jnp.float32),
            pltpu.VMEM((CH, D), jnp.float32),
            pltpu.VMEM((2 * NCH, CH), jnp.int32),
            pltpu.SemaphoreType.DMA,
            pltpu.SemaphoreType.DMA,
            pltpu.SemaphoreType.DMA,
        ],
    )
    return fn(y_sorted, dstp_3d)


def _combine_body(y0_ref, y1_ref, gates_ref, onw_ref, wout_ref, o_ref):
    gts = gates_ref[...]
    out = gts[:, 0:1] * y0_ref[...] + gts[:, 1:2] * y1_ref[...]
    out = _rms_rows(out, onw_ref[...])
    o_ref[...] = lax.dot_general(out, wout_ref[...], (((1,), (1,)), ((), ())),
                                 preferred_element_type=jnp.float32)


def _combine(y_pair, gates, onw2, W_out):
    return pl.pallas_call(
        _combine_body,
        grid=(NTB,),
        in_specs=[
            pl.BlockSpec((TM, D), lambda i: (i, 0)),
            pl.BlockSpec((TM, D), lambda i: (i + NTB, 0)),
            pl.BlockSpec((TM, E), lambda i: (i, 0)),
            pl.BlockSpec((1, D), lambda i: (0, 0)),
            pl.BlockSpec((D, D), lambda i: (0, 0)),
        ],
        out_specs=pl.BlockSpec((TM, D), lambda i: (i, 0)),
        out_shape=jax.ShapeDtypeStruct((N, D), jnp.float32),
    )(y_pair, y_pair, gates, onw2, W_out)


def kernel(s, W_in, in_norm_w, W_router, expert_bias, Wg, Wu, Wd, exp_norm_w,
           out_norm_w, W_out):
    inw2 = in_norm_w.reshape(1, D)
    bias2 = expert_bias.reshape(1, E)
    onw2 = out_norm_w.reshape(1, D)
    h, logits = _inproj(s, W_in, inw2, W_router)
    gates, dst0, dst1, be, nbu = _router(logits, bias2)
    be1 = be.reshape(NB)
    nbu1 = nbu.reshape(1)
    dst0_3d = dst0.reshape(NW, NCH, CH)
    dst1_3d = dst1.reshape(NW, NCH, CH)
    dstp_3d = jnp.concatenate([dst0.reshape(N), dst1.reshape(N)]).reshape(
        NW, 2 * NCH, CH)
    h_sorted = _sc_scatter(h, dst0_3d, dst1_3d)
    y_sorted = _ffn(be1, nbu1, h_sorted, Wg, Wu, Wd, exp_norm_w)
    y_pair = _sc_gather(y_sorted, dstp_3d)
    return _combine(y_pair, gates, onw2, W_out)


# speedup vs baseline: 1.0146x; 1.0146x over previous
import functools

import jax
import jax.numpy as jnp
from jax import lax
from jax.experimental import pallas as pl
from jax.experimental.pallas import tpu as pltpu
from jax.experimental.pallas import tpu_sc as plsc

N = 4096
D = 1024
DFF = 2048
E = 8
TOPK = 2
TM = 256
TMF = 384
NB = 29

P = NB * TMF
NTB = N // TM
NW = 32
TPW = N // NW
CH = 64
NCH = TPW // CH
EPS = 1e-6


def _rms_rows(x, w):
    return x * lax.rsqrt(jnp.mean(x * x, axis=-1, keepdims=True) + EPS) * w


def _inproj_body(s_ref, win_ref, inw_ref, wr_ref, h_ref, log_ref):
    h = lax.dot_general(s_ref[...], win_ref[...], (((1,), (1,)), ((), ())),
                        preferred_element_type=jnp.float32)
    h = _rms_rows(h, inw_ref[...])
    h_ref[...] = h
    log_ref[...] = lax.dot_general(h, wr_ref[...], (((1,), (1,)), ((), ())),
                                   preferred_element_type=jnp.float32)


def _inproj(s, W_in, inw2, W_router):
    return pl.pallas_call(
        _inproj_body,
        grid=(NTB,),
        in_specs=[
            pl.BlockSpec((TM, D), lambda i: (i, 0)),
            pl.BlockSpec((D, D), lambda i: (0, 0)),
            pl.BlockSpec((1, D), lambda i: (0, 0)),
            pl.BlockSpec((E, D), lambda i: (0, 0)),
        ],
        out_specs=[
            pl.BlockSpec((TM, D), lambda i: (i, 0)),
            pl.BlockSpec((TM, E), lambda i: (i, 0)),
        ],
        out_shape=[
            jax.ShapeDtypeStruct((N, D), jnp.float32),
            jax.ShapeDtypeStruct((N, E), jnp.float32),
        ],
    )(s, W_in, inw2, W_router)


def _router_body(log_ref, bias_ref, gates_ref, dst0_ref, dst1_ref, be_ref,
                 nbu_ref):
    logits = log_ref[...]
    biased = logits + bias_ref[...]
    iota_e = lax.broadcasted_iota(jnp.int32, (N, E), 1).astype(jnp.float32)
    max0 = jnp.max(biased, axis=1, keepdims=True)
    idx0 = jnp.min(jnp.where(biased == max0, iota_e, jnp.float32(E)),
                   axis=1, keepdims=True)
    masked = jnp.where(iota_e == idx0, -jnp.inf, biased)
    max1 = jnp.max(masked, axis=1, keepdims=True)
    idx1 = jnp.min(jnp.where(masked == max1, iota_e, jnp.float32(E)),
                   axis=1, keepdims=True)
    sel0 = jnp.sum(jnp.where(iota_e == idx0, logits, 0.0), axis=1, keepdims=True)
    sel1 = jnp.sum(jnp.where(iota_e == idx1, logits, 0.0), axis=1, keepdims=True)
    m = jnp.maximum(sel0, sel1)
    e0 = jnp.exp(sel0 - m)
    e1 = jnp.exp(sel1 - m)
    tot = e0 + e1
    g0 = e0 / tot
    g1 = e1 / tot
    gates_ref[...] = jnp.concatenate(
        [g0, g1, jnp.zeros((N, E - TOPK), jnp.float32)], axis=1)

    oh0 = (iota_e == idx0).astype(jnp.float32)
    oh1 = (iota_e == idx1).astype(jnp.float32)
    ohp = jnp.concatenate([oh0, oh1], axis=0)
    x = ohp
    sh = 1
    while sh < 2 * N:
        x = x + jnp.concatenate(
            [jnp.zeros((sh, E), jnp.float32), x[:2 * N - sh]], axis=0)
        sh *= 2
    excl = x - ohp
    counts = x[2 * N - 1:2 * N, :]
    blocks_e = jnp.floor((counts + (TMF - 1)) / TMF)
    b = blocks_e
    for shw in (1, 2, 4):
        b = b + jnp.concatenate(
            [jnp.zeros((1, shw), jnp.float32), b[:, :E - shw]], axis=1)
    bstart = b - blocks_e
    dstcol = excl + bstart * TMF
    dst_p = jnp.sum(ohp * dstcol, axis=1, keepdims=True).astype(jnp.int32)
    dst0_ref[...] = dst_p[:N]
    dst1_ref[...] = dst_p[N:]
    biot = lax.broadcasted_iota(jnp.int32, (NB, E), 0).astype(jnp.float32)
    be = jnp.sum((biot >= bstart).astype(jnp.float32), axis=1, keepdims=True) - 1.0
    be_ref[...] = jnp.clip(be, 0.0, float(E - 1)).astype(jnp.int32)
    nbu_ref[...] = jnp.sum(blocks_e, axis=1, keepdims=True).astype(jnp.int32)


def _router(logits, bias2):
    return pl.pallas_call(
        _router_body,
        grid=(1,),
        in_specs=[
            pl.BlockSpec((N, E), lambda i: (0, 0)),
            pl.BlockSpec((1, E), lambda i: (0, 0)),
        ],
        out_specs=[
            pl.BlockSpec((N, E), lambda i: (0, 0)),
            pl.BlockSpec((N, 1), lambda i: (0, 0)),
            pl.BlockSpec((N, 1), lambda i: (0, 0)),
            pl.BlockSpec((NB, 1), lambda i: (0, 0)),
            pl.BlockSpec((1, 1), lambda i: (0, 0)),
        ],
        out_shape=[
            jax.ShapeDtypeStruct((N, E), jnp.float32),
            jax.ShapeDtypeStruct((N, 1), jnp.int32),
            jax.ShapeDtypeStruct((N, 1), jnp.int32),
            jax.ShapeDtypeStruct((NB, 1), jnp.int32),
            jax.ShapeDtypeStruct((1, 1), jnp.int32),
        ],
    )(logits, bias2)


def _sc_scatter_body(h_hbm, dst0_hbm, dst1_hbm, out_hbm, rows_v,
                     idx0_v, idx1_v, sem):
    wid = lax.axis_index("s") * 2 + lax.axis_index("c")
    pltpu.sync_copy(dst0_hbm.at[wid], idx0_v)
    pltpu.sync_copy(dst1_hbm.at[wid], idx1_v)
    for j in range(NCH):
        t0 = wid * TPW + j * CH
        pltpu.sync_copy(h_hbm.at[pl.ds(t0, CH)], rows_v)
        w0 = pltpu.async_copy(rows_v, out_hbm.at[idx0_v.at[j]], sem)
        w1 = pltpu.async_copy(rows_v, out_hbm.at[idx1_v.at[j]], sem)
        w0.wait()
        w1.wait()


def _sc_scatter(h, dst0_3d, dst1_3d):
    mesh = plsc.VectorSubcoreMesh(core_axis_name="c", subcore_axis_name="s",
                                  num_cores=2, num_subcores=16)
    fn = pl.kernel(
        _sc_scatter_body,
        out_type=jax.ShapeDtypeStruct((P, D), jnp.float32),
        mesh=mesh,
        scratch_types=[
            pltpu.VMEM((CH, D), jnp.float32),
            pltpu.VMEM((NCH, CH), jnp.int32),
            pltpu.VMEM((NCH, CH), jnp.int32),
            pltpu.SemaphoreType.DMA,
        ],
    )
    return fn(h, dst0_3d, dst1_3d)


def _ffn_body(be_ref, nbu_ref, h_ref, wg_ref, wu_ref, wd_ref, enw_ref, y_ref):
    @pl.when(pl.program_id(0) < nbu_ref[0])
    def _():
        hb = h_ref[...]
        g = lax.dot_general(hb, wg_ref[0], (((1,), (1,)), ((), ())),
                            preferred_element_type=jnp.float32)
        u = lax.dot_general(hb, wu_ref[0], (((1,), (1,)), ((), ())),
                            preferred_element_type=jnp.float32)
        act = g * jax.nn.sigmoid(g) * u
        dd = lax.dot_general(act, wd_ref[0], (((1,), (1,)), ((), ())),
                             preferred_element_type=jnp.float32)
        y_ref[...] = _rms_rows(dd, enw_ref[0])


def _ffn(be, nbu, h_sorted, Wg, Wu, Wd, exp_norm_w):
    grid_spec = pltpu.PrefetchScalarGridSpec(
        num_scalar_prefetch=2,
        grid=(NB,),
        in_specs=[
            pl.BlockSpec((TMF, D), lambda b, eref, nref: (b, 0)),
            pl.BlockSpec((1, DFF, D), lambda b, eref, nref: (eref[b], 0, 0)),
            pl.BlockSpec((1, DFF, D), lambda b, eref, nref: (eref[b], 0, 0)),
            pl.BlockSpec((1, D, DFF), lambda b, eref, nref: (eref[b], 0, 0)),
            pl.BlockSpec((1, 1, D), lambda b, eref, nref: (eref[b], 0, 0)),
        ],
        out_specs=pl.BlockSpec((TMF, D), lambda b, eref, nref: (b, 0)),
    )
    return pl.pallas_call(
        _ffn_body,
        grid_spec=grid_spec,
        out_shape=jax.ShapeDtypeStruct((P, D), jnp.float32),
        compiler_params=pltpu.CompilerParams(
            dimension_semantics=("arbitrary",)),
    )(be, nbu, h_sorted, Wg, Wu, Wd, exp_norm_w.reshape(E, 1, D))


def _sc_gather_body(y_hbm, dstp_hbm, out_hbm, rows_v, idx_v, sem):
    wid = lax.axis_index("s") * 2 + lax.axis_index("c")
    pltpu.sync_copy(dstp_hbm.at[wid], idx_v)
    for j in range(2 * NCH):
        pltpu.async_copy(y_hbm.at[idx_v.at[j]], rows_v, sem).wait()
        pltpu.sync_copy(rows_v, out_hbm.at[pl.ds(wid * 2 * TPW + j * CH, CH)])


def _sc_gather(y_sorted, dstp_3d):
    mesh = plsc.VectorSubcoreMesh(core_axis_name="c", subcore_axis_name="s",
                                  num_cores=2, num_subcores=16)
    fn = pl.kernel(
        _sc_gather_body,
        out_type=jax.ShapeDtypeStruct((TOPK * N, D), jnp.float32),
        mesh=mesh,
        scratch_types=[
            pltpu.VMEM((CH, D), jnp.float32),
            pltpu.VMEM((2 * NCH, CH), jnp.int32),
            pltpu.SemaphoreType.DMA,
        ],
    )
    return fn(y_sorted, dstp_3d)


def _combine_body(y0_ref, y1_ref, gates_ref, onw_ref, wout_ref, o_ref):
    gts = gates_ref[...]
    out = gts[:, 0:1] * y0_ref[...] + gts[:, 1:2] * y1_ref[...]
    out = _rms_rows(out, onw_ref[...])
    o_ref[...] = lax.dot_general(out, wout_ref[...], (((1,), (1,)), ((), ())),
                                 preferred_element_type=jnp.float32)


def _combine(y_pair, gates, onw2, W_out):
    return pl.pallas_call(
        _combine_body,
        grid=(NTB,),
        in_specs=[
            pl.BlockSpec((TM, D), lambda i: (i, 0)),
            pl.BlockSpec((TM, D), lambda i: (i + NTB, 0)),
            pl.BlockSpec((TM, E), lambda i: (i, 0)),
            pl.BlockSpec((1, D), lambda i: (0, 0)),
            pl.BlockSpec((D, D), lambda i: (0, 0)),
        ],
        out_specs=pl.BlockSpec((TM, D), lambda i: (i, 0)),
        out_shape=jax.ShapeDtypeStruct((N, D), jnp.float32),
    )(y_pair, y_pair, gates, onw2, W_out)


def kernel(s, W_in, in_norm_w, W_router, expert_bias, Wg, Wu, Wd, exp_norm_w,
           out_norm_w, W_out):
    inw2 = in_norm_w.reshape(1, D)
    bias2 = expert_bias.reshape(1, E)
    onw2 = out_norm_w.reshape(1, D)
    h, logits = _inproj(s, W_in, inw2, W_router)
    gates, dst0, dst1, be, nbu = _router(logits, bias2)
    be1 = be.reshape(NB)
    nbu1 = nbu.reshape(1)
    dst0_3d = dst0.reshape(NW, NCH, CH)
    dst1_3d = dst1.reshape(NW, NCH, CH)
    dstp_3d = jnp.concatenate([dst0.reshape(N), dst1.reshape(N)]).reshape(
        NW, 2 * NCH, CH)
    h_sorted = _sc_scatter(h, dst0_3d, dst1_3d)
    y_sorted = _ffn(be1, nbu1, h_sorted, Wg, Wu, Wd, exp_norm_w)
    y_pair = _sc_gather(y_sorted, dstp_3d)
    return _combine(y_pair, gates, onw2, W_out)
